# baseline (speedup 1.0000x reference)
import functools

import jax
import jax.numpy as jnp
from jax import lax
from jax.experimental import pallas as pl
from jax.experimental.pallas import tpu as pltpu
from jax.experimental.pallas import tpu_sc as plsc

_TOKENS = 16384
_PATHS = 64
_NC = 2
_NS = 16
_NW = _NC * _NS
_ROWS_PER_W = _TOKENS // _NW

_mesh = plsc.VectorSubcoreMesh(core_axis_name="c", subcore_axis_name="s")


@functools.partial(
    pl.kernel,
    mesh=_mesh,
    out_type=jax.ShapeDtypeStruct((_TOKENS, _PATHS), jnp.int32),
    scratch_types=[
        pltpu.VMEM((_ROWS_PER_W, _PATHS), jnp.float32),
        pltpu.VMEM((_ROWS_PER_W, _PATHS), jnp.int32),
    ],
    compiler_params=pltpu.CompilerParams(needs_layout_passes=False),
)
def _sc_kernel(score_hbm, out_hbm, in_v, out_v):
    wid = lax.axis_index("s") * _NC + lax.axis_index("c")
    base = wid * _ROWS_PER_W
    pltpu.sync_copy(score_hbm.at[pl.ds(base, _ROWS_PER_W)], in_v)

    lane = lax.iota(jnp.int32, 16)
    lane0 = lane == 0
    ones = jnp.ones((16,), jnp.int32)
    zeros = jnp.zeros((16,), jnp.int32)

    def body(r, carry):
        vals = [in_v[r, pl.ds(g * 16, 16)] for g in range(4)]
        ps = [v > 0.0 for v in vals]
        vm = jnp.maximum(
            jnp.maximum(vals[0], vals[1]), jnp.maximum(vals[2], vals[3])
        )
        rmax = lax.reduce_max_p.bind(vm, axes=(0,))
        resid = (rmax <= 0.0) & lane0
        out_v[r, pl.ds(0, 16)] = (
            jnp.where(ps[0], ones, zeros) + jnp.where(resid, ones, zeros)
        )
        for g in range(1, 4):
            out_v[r, pl.ds(g * 16, 16)] = jnp.where(ps[g], ones, zeros)
        return carry

    lax.fori_loop(0, _ROWS_PER_W, body, jnp.int32(0))

    pltpu.sync_copy(out_v, out_hbm.at[pl.ds(base, _ROWS_PER_W)])


def kernel(score):
    return _sc_kernel(score)


# speedup vs baseline: 1.6693x; 1.6693x over previous
import jax
import jax.numpy as jnp
from jax.experimental import pallas as pl
from jax.experimental.pallas import tpu as pltpu

_TOKENS = 16384
_PATHS = 64


def _body(s_hbm, o_hbm, s_v, o_v, sem_in, sem_out):
    cin = pltpu.make_async_copy(s_hbm, s_v, sem_in)
    cin.start()
    cin.wait()
    s = s_v[...]
    pos = s > 0.0
    col = jax.lax.broadcasted_iota(jnp.int32, s.shape, 1)
    rmax = jnp.max(s, axis=1, keepdims=True)
    resid = (col == 0) & (rmax <= 0.0)
    o_v[...] = jnp.where(pos | resid, 1, 0).astype(jnp.int32)
    cout = pltpu.make_async_copy(o_v, o_hbm, sem_out)
    cout.start()
    cout.wait()


def kernel(score):
    return pl.pallas_call(
        _body,
        out_shape=jax.ShapeDtypeStruct((_TOKENS, _PATHS), jnp.int32),
        in_specs=[pl.BlockSpec(memory_space=pl.ANY)],
        out_specs=pl.BlockSpec(memory_space=pl.ANY),
        scratch_shapes=[
            pltpu.VMEM((_TOKENS, _PATHS), jnp.float32),
            pltpu.VMEM((_TOKENS, _PATHS), jnp.int32),
            pltpu.SemaphoreType.DMA,
            pltpu.SemaphoreType.DMA,
        ],
    )(score)
